# Initial kernel scaffold; baseline (speedup 1.0000x reference)
import jax
import jax.numpy as jnp
from jax.experimental import pallas as pl


def kernel(x, edge_attr, W_in, b_in, W_node, b_node, W_edge, b_edge, W_msg, b_msg, ln_g, ln_b, edge_index):
    raise NotImplementedError("write your pallas kernel here")


# speedup vs baseline: 2.8128x; 2.8128x over previous
import functools

import jax
import jax.numpy as jnp
from jax import lax
from jax.experimental import pallas as pl
from jax.experimental.pallas import tpu as pltpu
from jax.experimental.pallas import tpu_sc as plsc

LANE = 16
NCORES = 2
NSUB = 16
NW = NCORES * NSUB


def _mm_bias_body(x_ref, w_ref, b_ref, o_ref):
    o_ref[...] = (
        jnp.dot(x_ref[...], w_ref[...], preferred_element_type=jnp.float32)
        + b_ref[...]
    )


def _mm_bias(x, w, b, block_rows=2000):
    n, d = x.shape
    h = w.shape[1]
    return pl.pallas_call(
        _mm_bias_body,
        grid=(n // block_rows,),
        in_specs=[
            pl.BlockSpec((block_rows, d), lambda i: (i, 0)),
            pl.BlockSpec((d, h), lambda i: (0, 0)),
            pl.BlockSpec((1, h), lambda i: (0, 0)),
        ],
        out_specs=pl.BlockSpec((block_rows, h), lambda i: (i, 0)),
        out_shape=jax.ShapeDtypeStruct((n, h), jnp.float32),
    )(x, w, b.reshape(1, h))


def _ln_agg_body(h_ref, a0_ref, a1_ref, g_ref, b_ref, o_ref):
    s = h_ref[...] + a0_ref[...] + a1_ref[...]
    mu = jnp.mean(s, axis=-1, keepdims=True)
    d = s - mu
    var = jnp.mean(d * d, axis=-1, keepdims=True)
    o_ref[...] = d * lax.rsqrt(var + 1e-5) * g_ref[...] + b_ref[...]


def _ln_agg(h, a0, a1, g, b, block_rows=2000):
    n, hd = h.shape
    return pl.pallas_call(
        _ln_agg_body,
        grid=(n // block_rows,),
        in_specs=[
            pl.BlockSpec((block_rows, hd), lambda i: (i, 0)),
            pl.BlockSpec((block_rows, hd), lambda i: (i, 0)),
            pl.BlockSpec((block_rows, hd), lambda i: (i, 0)),
            pl.BlockSpec((1, hd), lambda i: (0, 0)),
            pl.BlockSpec((1, hd), lambda i: (0, 0)),
        ],
        out_specs=pl.BlockSpec((block_rows, hd), lambda i: (i, 0)),
        out_shape=jax.ShapeDtypeStruct((n, hd), jnp.float32),
    )(h, a0, a1, g.reshape(1, hd), b.reshape(1, hd))


@functools.lru_cache(maxsize=None)
def _make_sc_edge(n, e, h, ed, chunk):
    ew = e // NW
    nchunks = ew // chunk
    zrows = 128
    npad = ((n + NSUB * zrows - 1) // (NSUB * zrows)) * (NSUB * zrows)
    rows_per_tile = npad // NSUB
    ndrain = rows_per_tile // zrows
    nk = h // LANE

    mesh = plsc.VectorSubcoreMesh(core_axis_name="c", subcore_axis_name="s")

    @functools.partial(
        pl.kernel,
        out_type=jax.ShapeDtypeStruct((NCORES, npad, h), jnp.float32),
        mesh=mesh,
        scratch_types=[
            pltpu.VMEM_SHARED((npad, h), jnp.float32),
            pltpu.VMEM((chunk, h), jnp.float32),
            pltpu.VMEM((chunk,), jnp.int32),
            pltpu.VMEM((chunk,), jnp.int32),
            pltpu.VMEM((chunk * ed,), jnp.float32),
            pltpu.VMEM((ed, h), jnp.float32),
            pltpu.VMEM((zrows, h), jnp.float32),
            pltpu.SemaphoreType.DMA,
        ],
    )
    def sc_edge(a_hbm, ea_hbm, src_hbm, dst_hbm, w2_hbm, out_hbm,
                acc, rows_v, src_v, dst_v, ea_v, w2_v, zbuf, gsem):
        cid = lax.axis_index("c")
        sid = lax.axis_index("s")
        wid = cid * NSUB + sid

        zvec = jnp.zeros((LANE,), jnp.float32)

        def zrow(i, _):
            for k in range(nk):
                zbuf[i, pl.ds(k * LANE, LANE)] = zvec
            return 0

        lax.fori_loop(0, zrows, zrow, 0)
        for s in range(NSUB):
            @pl.when(sid == s)
            def _zero(s=s):
                for t in range(ndrain):
                    pltpu.sync_copy(zbuf, acc.at[pl.ds(s * rows_per_tile + t * zrows, zrows)])

        pltpu.sync_copy(w2_hbm, w2_v)
        w2c = [[w2_v[j, pl.ds(k * LANE, LANE)] for k in range(nk)]
               for j in range(ed)]

        plsc.subcore_barrier()

        def chunk_body(ci, _):
            base = wid * ew + ci * chunk
            pltpu.sync_copy(src_hbm.at[pl.ds(base, chunk)], src_v)
            pltpu.sync_copy(dst_hbm.at[pl.ds(base, chunk)], dst_v)
            pltpu.sync_copy(ea_hbm.at[pl.ds(base * ed, chunk * ed)], ea_v)
            pltpu.async_copy(a_hbm.at[src_v], rows_v, gsem).wait()

            epg = LANE // ed

            def group(gi, _):
                v = ea_v[pl.ds(gi * LANE, LANE)]
                for u in range(epg):
                    ei = gi * epg + u
                    a = [v[u * ed + j] for j in range(ed)]
                    for k in range(nk):
                        r = rows_v[ei, pl.ds(k * LANE, LANE)]
                        for j in range(ed):
                            r = r + a[j] * w2c[j][k]
                        rows_v[ei, pl.ds(k * LANE, LANE)] = jnp.maximum(r, 0.0)
                return 0

            lax.fori_loop(0, (chunk * ed) // LANE, group, 0)
            pltpu.sync_copy(rows_v, acc.at[dst_v], add=True)
            return 0

        lax.fori_loop(0, nchunks, chunk_body, 0)

        plsc.subcore_barrier()

        for s in range(NSUB):
            @pl.when(sid == s)
            def _drain(s=s):
                for t in range(ndrain):
                    start = s * rows_per_tile + t * zrows
                    pltpu.sync_copy(acc.at[pl.ds(start, zrows)], zbuf)
                    pltpu.sync_copy(zbuf, out_hbm.at[cid, pl.ds(start, zrows)])

    return sc_edge


def kernel(x, edge_attr, W_in, b_in, W_node, b_node, W_edge, b_edge,
           W_msg, b_msg, ln_g, ln_b, edge_index):
    n, _ = x.shape
    e = edge_index.shape[1]
    ed = edge_attr.shape[1]
    hd = W_in.shape[1]
    nlayers = W_node.shape[0]

    wtop = W_msg[:, :hd, :]
    wbot = W_msg[:, hd:, :]
    wc = jnp.einsum("lij,ljk->lik", W_node, wtop)
    beta = (jnp.einsum("lj,ljk->lk", b_node, wtop)
            + jnp.einsum("lj,ljk->lk", b_edge, wbot) + b_msg)
    w2 = jnp.einsum("lij,ljk->lik", W_edge, wbot)

    src = edge_index[0]
    dst = edge_index[1]

    sc_edge = _make_sc_edge(n, e, hd, ed, 80)

    h = _mm_bias(x, W_in, b_in)
    for l in range(nlayers):
        a = _mm_bias(h, wc[l], beta[l])
        parts = sc_edge(a, edge_attr.reshape(-1), src, dst, w2[l])
        h = _ln_agg(h, parts[0, :n], parts[1, :n], ln_g[l], ln_b[l])
    return h


# speedup vs baseline: 3.5253x; 1.2533x over previous
import functools

import jax
import jax.numpy as jnp
from jax import lax
from jax.experimental import pallas as pl
from jax.experimental.pallas import tpu as pltpu
from jax.experimental.pallas import tpu_sc as plsc

LANE = 16
NCORES = 2
NSUB = 16
NW = NCORES * NSUB


def _mm_bias_body(x_ref, w_ref, b_ref, o_ref):
    o_ref[...] = (
        jnp.dot(x_ref[...], w_ref[...], preferred_element_type=jnp.float32)
        + b_ref[...]
    )


def _mm_bias(x, w, b, block_rows=2000):
    n, d = x.shape
    h = w.shape[1]
    return pl.pallas_call(
        _mm_bias_body,
        grid=(n // block_rows,),
        in_specs=[
            pl.BlockSpec((block_rows, d), lambda i: (i, 0)),
            pl.BlockSpec((d, h), lambda i: (0, 0)),
            pl.BlockSpec((1, h), lambda i: (0, 0)),
        ],
        out_specs=pl.BlockSpec((block_rows, h), lambda i: (i, 0)),
        out_shape=jax.ShapeDtypeStruct((n, h), jnp.float32),
    )(x, w, b.reshape(1, h))


def _ln_agg_body(h_ref, a0_ref, a1_ref, g_ref, b_ref, o_ref):
    s = h_ref[...] + a0_ref[...] + a1_ref[...]
    mu = jnp.mean(s, axis=-1, keepdims=True)
    d = s - mu
    var = jnp.mean(d * d, axis=-1, keepdims=True)
    o_ref[...] = d * lax.rsqrt(var + 1e-5) * g_ref[...] + b_ref[...]


def _ln_agg(h, a0, a1, g, b, block_rows=2000):
    n, hd = h.shape
    return pl.pallas_call(
        _ln_agg_body,
        grid=(n // block_rows,),
        in_specs=[
            pl.BlockSpec((block_rows, hd), lambda i: (i, 0)),
            pl.BlockSpec((block_rows, hd), lambda i: (i, 0)),
            pl.BlockSpec((block_rows, hd), lambda i: (i, 0)),
            pl.BlockSpec((1, hd), lambda i: (0, 0)),
            pl.BlockSpec((1, hd), lambda i: (0, 0)),
        ],
        out_specs=pl.BlockSpec((block_rows, hd), lambda i: (i, 0)),
        out_shape=jax.ShapeDtypeStruct((n, hd), jnp.float32),
    )(h, a0, a1, g.reshape(1, hd), b.reshape(1, hd))


@functools.lru_cache(maxsize=None)
def _make_sc_edge(n, e, h, ed, chunk):
    ew = e // NW
    nchunks = ew // chunk
    zrows = 128
    npad = ((n + NSUB * zrows - 1) // (NSUB * zrows)) * (NSUB * zrows)
    rows_per_tile = npad // NSUB
    ndrain = rows_per_tile // zrows
    nk = h // LANE

    mesh = plsc.VectorSubcoreMesh(core_axis_name="c", subcore_axis_name="s")

    @functools.partial(
        pl.kernel,
        out_type=jax.ShapeDtypeStruct((NCORES, npad, h), jnp.float32),
        mesh=mesh,
        scratch_types=[
            pltpu.VMEM_SHARED((npad, h), jnp.float32),
            pltpu.VMEM((chunk, h), jnp.float32),
            pltpu.VMEM((chunk, h), jnp.float32),
            pltpu.VMEM((chunk,), jnp.int32),
            pltpu.VMEM((chunk,), jnp.int32),
            pltpu.VMEM((chunk,), jnp.int32),
            pltpu.VMEM((chunk,), jnp.int32),
            pltpu.VMEM((chunk * ed,), jnp.float32),
            pltpu.VMEM((chunk * ed,), jnp.float32),
            pltpu.VMEM((ed, h), jnp.float32),
            pltpu.VMEM((zrows, h), jnp.float32),
            pltpu.SemaphoreType.DMA,
            pltpu.SemaphoreType.DMA,
            pltpu.SemaphoreType.DMA,
            pltpu.SemaphoreType.DMA,
        ],
    )
    def sc_edge(a_hbm, ea_hbm, src_hbm, dst_hbm, w2_hbm, out_hbm,
                acc, rows0, rows1, src0, src1, dst0, dst1, ea0, ea1, w2_v,
                zbuf, gsem0, gsem1, ssem0, ssem1):
        rows_v = (rows0, rows1)
        src_v = (src0, src1)
        dst_v = (dst0, dst1)
        ea_v = (ea0, ea1)
        gsems = (gsem0, gsem1)
        ssems = (ssem0, ssem1)
        cid = lax.axis_index("c")
        sid = lax.axis_index("s")
        wid = cid * NSUB + sid

        zvec = jnp.zeros((LANE,), jnp.float32)

        def zrow(i, _):
            for k in range(nk):
                zbuf[i, pl.ds(k * LANE, LANE)] = zvec
            return 0

        lax.fori_loop(0, zrows, zrow, 0)
        for s in range(NSUB):
            @pl.when(sid == s)
            def _zero(s=s):
                for t in range(ndrain):
                    pltpu.sync_copy(zbuf, acc.at[pl.ds(s * rows_per_tile + t * zrows, zrows)])

        pltpu.sync_copy(w2_hbm, w2_v)
        w2c = [[w2_v[j, pl.ds(k * LANE, LANE)] for k in range(nk)]
               for j in range(ed)]

        epg = LANE // ed

        def prep(ci, b):
            base = wid * ew + ci * chunk
            pltpu.sync_copy(src_hbm.at[pl.ds(base, chunk)], src_v[b])
            pltpu.sync_copy(dst_hbm.at[pl.ds(base, chunk)], dst_v[b])
            pltpu.sync_copy(ea_hbm.at[pl.ds(base * ed, chunk * ed)], ea_v[b])
            pltpu.async_copy(a_hbm.at[src_v[b]], rows_v[b], gsems[b])

        def wait_gather(b):
            pltpu.make_async_copy(a_hbm.at[src_v[b]], rows_v[b],
                                  gsems[b]).wait()

        def compute(b):
            def group(gi, _):
                v = ea_v[b][pl.ds(gi * LANE, LANE)]
                for u in range(epg):
                    ei = gi * epg + u
                    a = [v[u * ed + j] for j in range(ed)]
                    for k in range(nk):
                        r = rows_v[b][ei, pl.ds(k * LANE, LANE)]
                        for j in range(ed):
                            r = r + a[j] * w2c[j][k]
                        rows_v[b][ei, pl.ds(k * LANE, LANE)] = jnp.maximum(r, 0.0)
                return 0
            lax.fori_loop(0, (chunk * ed) // LANE, group, 0)

        def scatter(b):
            pltpu.async_copy(rows_v[b], acc.at[dst_v[b]], ssems[b],
                             add=True)

        def wait_scatter(b):
            pltpu.make_async_copy(rows_v[b], acc.at[dst_v[b]],
                                  ssems[b]).wait()

        zidx16 = lax.iota(jnp.int32, LANE) * 0
        for j in range(chunk // LANE):
            dst_v[1][pl.ds(j * LANE, LANE)] = zidx16
        pltpu.async_copy(zbuf.at[pl.ds(0, chunk)], acc.at[dst_v[1]],
                         ssems[1], add=True)

        plsc.subcore_barrier()

        prep(0, 0)

        def pair_body(i, _):
            cA = 2 * i
            wait_gather(0)
            wait_scatter(1)
            prep(cA + 1, 1)
            compute(0)
            scatter(0)
            wait_gather(1)
            wait_scatter(0)
            prep(cA + 2, 0)
            compute(1)
            scatter(1)
            return 0

        lax.fori_loop(0, (nchunks - 1) // 2, pair_body, 0)

        wait_gather(0)
        wait_scatter(1)
        compute(0)
        scatter(0)
        wait_scatter(0)

        plsc.subcore_barrier()

        for s in range(NSUB):
            @pl.when(sid == s)
            def _drain(s=s):
                for t in range(ndrain):
                    start = s * rows_per_tile + t * zrows
                    pltpu.sync_copy(acc.at[pl.ds(start, zrows)], zbuf)
                    pltpu.sync_copy(zbuf, out_hbm.at[cid, pl.ds(start, zrows)])

    return sc_edge


def kernel(x, edge_attr, W_in, b_in, W_node, b_node, W_edge, b_edge,
           W_msg, b_msg, ln_g, ln_b, edge_index):
    n, _ = x.shape
    e = edge_index.shape[1]
    ed = edge_attr.shape[1]
    hd = W_in.shape[1]
    nlayers = W_node.shape[0]

    wtop = W_msg[:, :hd, :]
    wbot = W_msg[:, hd:, :]
    wc = jnp.einsum("lij,ljk->lik", W_node, wtop)
    beta = (jnp.einsum("lj,ljk->lk", b_node, wtop)
            + jnp.einsum("lj,ljk->lk", b_edge, wbot) + b_msg)
    w2 = jnp.einsum("lij,ljk->lik", W_edge, wbot)

    src = edge_index[0]
    dst = edge_index[1]

    sc_edge = _make_sc_edge(n, e, hd, ed, 80)

    h = _mm_bias(x, W_in, b_in)
    for l in range(nlayers):
        a = _mm_bias(h, wc[l], beta[l])
        parts = sc_edge(a, edge_attr.reshape(-1), src, dst, w2[l])
        h = _ln_agg(h, parts[0, :n], parts[1, :n], ln_g[l], ln_b[l])
    return h


# speedup vs baseline: 3.5626x; 1.0106x over previous
import functools

import jax
import jax.numpy as jnp
from jax import lax
from jax.experimental import pallas as pl
from jax.experimental.pallas import tpu as pltpu
from jax.experimental.pallas import tpu_sc as plsc

LANE = 16
NCORES = 2
NSUB = 16
NW = NCORES * NSUB


def _mm_bias_body(x_ref, w_ref, b_ref, o_ref):
    o_ref[...] = (
        jnp.dot(x_ref[...], w_ref[...], preferred_element_type=jnp.float32)
        + b_ref[...]
    )


def _mm_bias(x, w, b, block_rows=2000):
    n, d = x.shape
    h = w.shape[1]
    return pl.pallas_call(
        _mm_bias_body,
        grid=(n // block_rows,),
        in_specs=[
            pl.BlockSpec((block_rows, d), lambda i: (i, 0)),
            pl.BlockSpec((d, h), lambda i: (0, 0)),
            pl.BlockSpec((1, h), lambda i: (0, 0)),
        ],
        out_specs=pl.BlockSpec((block_rows, h), lambda i: (i, 0)),
        out_shape=jax.ShapeDtypeStruct((n, h), jnp.float32),
    )(x, w, b.reshape(1, h))


def _in_mm_body(x_ref, w0_ref, b0_ref, w1_ref, b1_ref, h_ref, a_ref):
    h = (jnp.dot(x_ref[...], w0_ref[...], preferred_element_type=jnp.float32)
         + b0_ref[...])
    h_ref[...] = h
    a_ref[...] = (jnp.dot(h, w1_ref[...], preferred_element_type=jnp.float32)
                  + b1_ref[...])


def _in_mm(x, w0, b0, w1, b1, block_rows=2000):
    n, d = x.shape
    hd = w0.shape[1]
    rspec = pl.BlockSpec((block_rows, d), lambda i: (i, 0))
    wspec = lambda r, c: pl.BlockSpec((r, c), lambda i: (0, 0))
    return pl.pallas_call(
        _in_mm_body,
        grid=(n // block_rows,),
        in_specs=[rspec, wspec(d, hd), wspec(1, hd), wspec(hd, hd), wspec(1, hd)],
        out_specs=[pl.BlockSpec((block_rows, hd), lambda i: (i, 0))] * 2,
        out_shape=[jax.ShapeDtypeStruct((n, hd), jnp.float32)] * 2,
    )(x, w0, b0.reshape(1, hd), w1, b1.reshape(1, hd))


def _ln_mm_body(h_ref, a0_ref, a1_ref, g_ref, b_ref, w_ref, beta_ref,
                hn_ref, a_ref):
    s = h_ref[...] + a0_ref[...] + a1_ref[...]
    mu = jnp.mean(s, axis=-1, keepdims=True)
    d = s - mu
    var = jnp.mean(d * d, axis=-1, keepdims=True)
    hn = d * lax.rsqrt(var + 1e-5) * g_ref[...] + b_ref[...]
    hn_ref[...] = hn
    a_ref[...] = (jnp.dot(hn, w_ref[...], preferred_element_type=jnp.float32)
                  + beta_ref[...])


def _ln_mm(h, a0, a1, g, b, w, beta, block_rows=2000):
    n, hd = h.shape
    rspec = pl.BlockSpec((block_rows, hd), lambda i: (i, 0))
    wspec = lambda r, c: pl.BlockSpec((r, c), lambda i: (0, 0))
    return pl.pallas_call(
        _ln_mm_body,
        grid=(n // block_rows,),
        in_specs=[rspec, rspec, rspec, wspec(1, hd), wspec(1, hd),
                  wspec(hd, hd), wspec(1, hd)],
        out_specs=[rspec] * 2,
        out_shape=[jax.ShapeDtypeStruct((n, hd), jnp.float32)] * 2,
    )(h, a0, a1, g.reshape(1, hd), b.reshape(1, hd), w, beta.reshape(1, hd))


def _ln_agg_body(h_ref, a0_ref, a1_ref, g_ref, b_ref, o_ref):
    s = h_ref[...] + a0_ref[...] + a1_ref[...]
    mu = jnp.mean(s, axis=-1, keepdims=True)
    d = s - mu
    var = jnp.mean(d * d, axis=-1, keepdims=True)
    o_ref[...] = d * lax.rsqrt(var + 1e-5) * g_ref[...] + b_ref[...]


def _ln_agg(h, a0, a1, g, b, block_rows=2000):
    n, hd = h.shape
    return pl.pallas_call(
        _ln_agg_body,
        grid=(n // block_rows,),
        in_specs=[
            pl.BlockSpec((block_rows, hd), lambda i: (i, 0)),
            pl.BlockSpec((block_rows, hd), lambda i: (i, 0)),
            pl.BlockSpec((block_rows, hd), lambda i: (i, 0)),
            pl.BlockSpec((1, hd), lambda i: (0, 0)),
            pl.BlockSpec((1, hd), lambda i: (0, 0)),
        ],
        out_specs=pl.BlockSpec((block_rows, hd), lambda i: (i, 0)),
        out_shape=jax.ShapeDtypeStruct((n, hd), jnp.float32),
    )(h, a0, a1, g.reshape(1, hd), b.reshape(1, hd))


@functools.lru_cache(maxsize=None)
def _make_sc_edge(n, e, h, ed, chunk):
    ew = e // NW
    nchunks = ew // chunk
    zrows = 128
    npad = ((n + NSUB * zrows - 1) // (NSUB * zrows)) * (NSUB * zrows)
    rows_per_tile = npad // NSUB
    ndrain = rows_per_tile // zrows
    nk = h // LANE

    mesh = plsc.VectorSubcoreMesh(core_axis_name="c", subcore_axis_name="s")

    @functools.partial(
        pl.kernel,
        out_type=jax.ShapeDtypeStruct((NCORES, npad, h), jnp.float32),
        mesh=mesh,
        scratch_types=[
            pltpu.VMEM_SHARED((npad, h), jnp.float32),
            pltpu.VMEM((chunk, h), jnp.float32),
            pltpu.VMEM((chunk, h), jnp.float32),
            pltpu.VMEM((chunk,), jnp.int32),
            pltpu.VMEM((chunk,), jnp.int32),
            pltpu.VMEM((chunk,), jnp.int32),
            pltpu.VMEM((chunk,), jnp.int32),
            pltpu.VMEM((chunk * ed,), jnp.float32),
            pltpu.VMEM((chunk * ed,), jnp.float32),
            pltpu.VMEM((ed, h), jnp.float32),
            pltpu.VMEM((zrows, h), jnp.float32),
            pltpu.SemaphoreType.DMA,
            pltpu.SemaphoreType.DMA,
            pltpu.SemaphoreType.DMA,
            pltpu.SemaphoreType.DMA,
        ],
    )
    def sc_edge(a_hbm, ea_hbm, src_hbm, dst_hbm, w2_hbm, out_hbm,
                acc, rows0, rows1, src0, src1, dst0, dst1, ea0, ea1, w2_v,
                zbuf, gsem0, gsem1, ssem0, ssem1):
        rows_v = (rows0, rows1)
        src_v = (src0, src1)
        dst_v = (dst0, dst1)
        ea_v = (ea0, ea1)
        gsems = (gsem0, gsem1)
        ssems = (ssem0, ssem1)
        cid = lax.axis_index("c")
        sid = lax.axis_index("s")
        wid = cid * NSUB + sid

        zvec = jnp.zeros((LANE,), jnp.float32)

        def zrow(i, _):
            for k in range(nk):
                zbuf[i, pl.ds(k * LANE, LANE)] = zvec
            return 0

        lax.fori_loop(0, zrows, zrow, 0)
        for s in range(NSUB):
            @pl.when(sid == s)
            def _zero(s=s):
                for t in range(ndrain):
                    pltpu.sync_copy(zbuf, acc.at[pl.ds(s * rows_per_tile + t * zrows, zrows)])

        pltpu.sync_copy(w2_hbm, w2_v)
        w2c = [[w2_v[j, pl.ds(k * LANE, LANE)] for k in range(nk)]
               for j in range(ed)]

        epg = LANE // ed

        def prep(ci, b):
            base = wid * ew + ci * chunk
            pltpu.sync_copy(src_hbm.at[pl.ds(base, chunk)], src_v[b])
            pltpu.sync_copy(dst_hbm.at[pl.ds(base, chunk)], dst_v[b])
            pltpu.sync_copy(ea_hbm.at[pl.ds(base * ed, chunk * ed)], ea_v[b])
            pltpu.async_copy(a_hbm.at[src_v[b]], rows_v[b], gsems[b])

        def wait_gather(b):
            pltpu.make_async_copy(a_hbm.at[src_v[b]], rows_v[b],
                                  gsems[b]).wait()

        def compute(b):
            def group(gi, _):
                v = ea_v[b][pl.ds(gi * LANE, LANE)]
                for u in range(epg):
                    ei = gi * epg + u
                    a = [v[u * ed + j] for j in range(ed)]
                    for k in range(nk):
                        r = rows_v[b][ei, pl.ds(k * LANE, LANE)]
                        for j in range(ed):
                            r = r + a[j] * w2c[j][k]
                        rows_v[b][ei, pl.ds(k * LANE, LANE)] = jnp.maximum(r, 0.0)
                return 0
            lax.fori_loop(0, (chunk * ed) // LANE, group, 0)

        def scatter(b):
            pltpu.async_copy(rows_v[b], acc.at[dst_v[b]], ssems[b],
                             add=True)

        def wait_scatter(b):
            pltpu.make_async_copy(rows_v[b], acc.at[dst_v[b]],
                                  ssems[b]).wait()

        zidx16 = lax.iota(jnp.int32, LANE) * 0
        for j in range(chunk // LANE):
            dst_v[1][pl.ds(j * LANE, LANE)] = zidx16
        pltpu.async_copy(zbuf.at[pl.ds(0, chunk)], acc.at[dst_v[1]],
                         ssems[1], add=True)

        plsc.subcore_barrier()

        prep(0, 0)

        def pair_body(i, _):
            cA = 2 * i
            wait_gather(0)
            wait_scatter(1)
            prep(cA + 1, 1)
            compute(0)
            scatter(0)
            wait_gather(1)
            wait_scatter(0)
            prep(cA + 2, 0)
            compute(1)
            scatter(1)
            return 0

        lax.fori_loop(0, (nchunks - 1) // 2, pair_body, 0)

        wait_gather(0)
        wait_scatter(1)
        compute(0)
        scatter(0)
        wait_scatter(0)

        plsc.subcore_barrier()

        for s in range(NSUB):
            @pl.when(sid == s)
            def _drain(s=s):
                for t in range(ndrain):
                    start = s * rows_per_tile + t * zrows
                    pltpu.sync_copy(acc.at[pl.ds(start, zrows)], zbuf)
                    pltpu.sync_copy(zbuf, out_hbm.at[cid, pl.ds(start, zrows)])

    return sc_edge


def kernel(x, edge_attr, W_in, b_in, W_node, b_node, W_edge, b_edge,
           W_msg, b_msg, ln_g, ln_b, edge_index):
    n, _ = x.shape
    e = edge_index.shape[1]
    ed = edge_attr.shape[1]
    hd = W_in.shape[1]
    nlayers = W_node.shape[0]

    wtop = W_msg[:, :hd, :]
    wbot = W_msg[:, hd:, :]
    wc = jnp.einsum("lij,ljk->lik", W_node, wtop)
    beta = (jnp.einsum("lj,ljk->lk", b_node, wtop)
            + jnp.einsum("lj,ljk->lk", b_edge, wbot) + b_msg)
    w2 = jnp.einsum("lij,ljk->lik", W_edge, wbot)

    src = edge_index[0]
    dst = edge_index[1]

    sc_edge = _make_sc_edge(n, e, hd, ed, 80)

    h, a = _in_mm(x, W_in, b_in, wc[0], beta[0])
    for l in range(nlayers):
        parts = sc_edge(a, edge_attr.reshape(-1), src, dst, w2[l])
        if l + 1 < nlayers:
            h, a = _ln_mm(h, parts[0, :n], parts[1, :n], ln_g[l], ln_b[l],
                          wc[l + 1], beta[l + 1])
        else:
            h = _ln_agg(h, parts[0, :n], parts[1, :n], ln_g[l], ln_b[l])
    return h


# speedup vs baseline: 4.4734x; 1.2556x over previous
import functools

import jax
import jax.numpy as jnp
from jax import lax
from jax.experimental import pallas as pl
from jax.experimental.pallas import tpu as pltpu
from jax.experimental.pallas import tpu_sc as plsc

LANE = 16
NCORES = 2
NSUB = 16
NW = NCORES * NSUB


def _mm_bias_body(x_ref, w_ref, b_ref, o_ref):
    o_ref[...] = (
        jnp.dot(x_ref[...], w_ref[...], preferred_element_type=jnp.float32)
        + b_ref[...]
    )


def _mm_bias(x, w, b, block_rows=2000):
    n, d = x.shape
    h = w.shape[1]
    return pl.pallas_call(
        _mm_bias_body,
        grid=(n // block_rows,),
        in_specs=[
            pl.BlockSpec((block_rows, d), lambda i: (i, 0)),
            pl.BlockSpec((d, h), lambda i: (0, 0)),
            pl.BlockSpec((1, h), lambda i: (0, 0)),
        ],
        out_specs=pl.BlockSpec((block_rows, h), lambda i: (i, 0)),
        out_shape=jax.ShapeDtypeStruct((n, h), jnp.float32),
    )(x, w, b.reshape(1, h))


def _in_mm_body(x_ref, w0_ref, b0_ref, w1_ref, b1_ref, h_ref, a_ref):
    h = (jnp.dot(x_ref[...], w0_ref[...], preferred_element_type=jnp.float32)
         + b0_ref[...])
    h_ref[...] = h
    a_ref[...] = (jnp.dot(h, w1_ref[...], preferred_element_type=jnp.float32)
                  + b1_ref[...])


def _in_mm(x, w0, b0, w1, b1, block_rows=2000):
    n, d = x.shape
    hd = w0.shape[1]
    rspec = pl.BlockSpec((block_rows, d), lambda i: (i, 0))
    wspec = lambda r, c: pl.BlockSpec((r, c), lambda i: (0, 0))
    return pl.pallas_call(
        _in_mm_body,
        grid=(n // block_rows,),
        in_specs=[rspec, wspec(d, hd), wspec(1, hd), wspec(hd, hd), wspec(1, hd)],
        out_specs=[pl.BlockSpec((block_rows, hd), lambda i: (i, 0))] * 2,
        out_shape=[jax.ShapeDtypeStruct((n, hd), jnp.float32)] * 2,
    )(x, w0, b0.reshape(1, hd), w1, b1.reshape(1, hd))


def _ln_mm_body(h_ref, a0_ref, a1_ref, g_ref, b_ref, w_ref, beta_ref,
                hn_ref, a_ref):
    s = h_ref[...] + a0_ref[...] + a1_ref[...]
    mu = jnp.mean(s, axis=-1, keepdims=True)
    d = s - mu
    var = jnp.mean(d * d, axis=-1, keepdims=True)
    hn = d * lax.rsqrt(var + 1e-5) * g_ref[...] + b_ref[...]
    hn_ref[...] = hn
    a_ref[...] = (jnp.dot(hn, w_ref[...], preferred_element_type=jnp.float32)
                  + beta_ref[...])


def _ln_mm(h, a0, a1, g, b, w, beta, block_rows=2000):
    n, hd = h.shape
    rspec = pl.BlockSpec((block_rows, hd), lambda i: (i, 0))
    wspec = lambda r, c: pl.BlockSpec((r, c), lambda i: (0, 0))
    return pl.pallas_call(
        _ln_mm_body,
        grid=(n // block_rows,),
        in_specs=[rspec, rspec, rspec, wspec(1, hd), wspec(1, hd),
                  wspec(hd, hd), wspec(1, hd)],
        out_specs=[rspec] * 2,
        out_shape=[jax.ShapeDtypeStruct((n, hd), jnp.float32)] * 2,
    )(h, a0, a1, g.reshape(1, hd), b.reshape(1, hd), w, beta.reshape(1, hd))


def _ln_agg_body(h_ref, a0_ref, a1_ref, g_ref, b_ref, o_ref):
    s = h_ref[...] + a0_ref[...] + a1_ref[...]
    mu = jnp.mean(s, axis=-1, keepdims=True)
    d = s - mu
    var = jnp.mean(d * d, axis=-1, keepdims=True)
    o_ref[...] = d * lax.rsqrt(var + 1e-5) * g_ref[...] + b_ref[...]


def _ln_agg(h, a0, a1, g, b, block_rows=2000):
    n, hd = h.shape
    return pl.pallas_call(
        _ln_agg_body,
        grid=(n // block_rows,),
        in_specs=[
            pl.BlockSpec((block_rows, hd), lambda i: (i, 0)),
            pl.BlockSpec((block_rows, hd), lambda i: (i, 0)),
            pl.BlockSpec((block_rows, hd), lambda i: (i, 0)),
            pl.BlockSpec((1, hd), lambda i: (0, 0)),
            pl.BlockSpec((1, hd), lambda i: (0, 0)),
        ],
        out_specs=pl.BlockSpec((block_rows, hd), lambda i: (i, 0)),
        out_shape=jax.ShapeDtypeStruct((n, hd), jnp.float32),
    )(h, a0, a1, g.reshape(1, hd), b.reshape(1, hd))


@functools.lru_cache(maxsize=None)
def _make_sc_edge(n, e, h, ed, chunk):
    ew = e // NW
    nchunks = ew // chunk
    zrows = 128
    npad = ((n + NSUB * zrows - 1) // (NSUB * zrows)) * (NSUB * zrows)
    rows_per_tile = npad // NSUB
    ndrain = rows_per_tile // zrows
    nk = h // LANE

    mesh = plsc.VectorSubcoreMesh(core_axis_name="c", subcore_axis_name="s")

    @functools.partial(
        pl.kernel,
        out_type=jax.ShapeDtypeStruct((NCORES, npad, h), jnp.float32),
        mesh=mesh,
        scratch_types=[
            pltpu.VMEM_SHARED((npad, h), jnp.float32),
            pltpu.VMEM((chunk, h), jnp.float32),
            pltpu.VMEM((chunk, h), jnp.float32),
            pltpu.VMEM((chunk,), jnp.int32),
            pltpu.VMEM((chunk,), jnp.int32),
            pltpu.VMEM((chunk,), jnp.int32),
            pltpu.VMEM((chunk,), jnp.int32),
            pltpu.VMEM((chunk * ed,), jnp.float32),
            pltpu.VMEM((chunk * ed,), jnp.float32),
            pltpu.VMEM((ed, h), jnp.float32),
            pltpu.VMEM((zrows, h), jnp.float32),
            pltpu.SemaphoreType.DMA,
            pltpu.SemaphoreType.DMA,
            pltpu.SemaphoreType.DMA,
            pltpu.SemaphoreType.DMA,
        ],
    )
    def sc_edge(a_hbm, ea_hbm, src_hbm, dst_hbm, w2_hbm, out_hbm,
                acc, rows0, rows1, src0, src1, dst0, dst1, ea0, ea1, w2_v,
                zbuf, gsem0, gsem1, ssem0, ssem1):
        rows_v = (rows0, rows1)
        src_v = (src0, src1)
        dst_v = (dst0, dst1)
        ea_v = (ea0, ea1)
        gsems = (gsem0, gsem1)
        ssems = (ssem0, ssem1)
        cid = lax.axis_index("c")
        sid = lax.axis_index("s")
        wid = cid * NSUB + sid

        zvec = jnp.zeros((LANE,), jnp.float32)

        def zrow(i, _):
            for k in range(nk):
                zbuf[i, pl.ds(k * LANE, LANE)] = zvec
            return 0

        lax.fori_loop(0, zrows, zrow, 0)
        for s in range(NSUB):
            @pl.when(sid == s)
            def _zero(s=s):
                for t in range(ndrain):
                    pltpu.sync_copy(zbuf, acc.at[pl.ds(s * rows_per_tile + t * zrows, zrows)])

        pltpu.sync_copy(w2_hbm, w2_v)
        w2c = [[w2_v[j, pl.ds(k * LANE, LANE)] for k in range(nk)]
               for j in range(ed)]

        epg = LANE // ed

        def prep(ci, b):
            base = wid * ew + ci * chunk
            d1 = pltpu.async_copy(src_hbm.at[pl.ds(base, chunk)], src_v[b],
                                  gsems[b])
            d2 = pltpu.async_copy(dst_hbm.at[pl.ds(base, chunk)], dst_v[b],
                                  gsems[b])
            d3 = pltpu.async_copy(ea_hbm.at[pl.ds(base * ed, chunk * ed)],
                                  ea_v[b], gsems[b])
            d1.wait()
            d2.wait()
            d3.wait()
            pltpu.async_copy(a_hbm.at[src_v[b]], rows_v[b], gsems[b])

        def wait_gather(b):
            pltpu.make_async_copy(a_hbm.at[src_v[b]], rows_v[b],
                                  gsems[b]).wait()

        def compute(b):
            def group(gi, _):
                v = ea_v[b][pl.ds(gi * LANE, LANE)]
                for u in range(epg):
                    ei = gi * epg + u
                    a = [v[u * ed + j] for j in range(ed)]
                    for k in range(nk):
                        r = rows_v[b][ei, pl.ds(k * LANE, LANE)]
                        for j in range(ed):
                            r = r + a[j] * w2c[j][k]
                        rows_v[b][ei, pl.ds(k * LANE, LANE)] = jnp.maximum(r, 0.0)
                return 0
            lax.fori_loop(0, (chunk * ed) // LANE, group, 0)

        def scatter(b):
            pltpu.async_copy(rows_v[b], acc.at[dst_v[b]], ssems[b],
                             add=True)

        def wait_scatter(b):
            pltpu.make_async_copy(rows_v[b], acc.at[dst_v[b]],
                                  ssems[b]).wait()

        zidx16 = lax.iota(jnp.int32, LANE) * 0
        for j in range(chunk // LANE):
            dst_v[1][pl.ds(j * LANE, LANE)] = zidx16
        pltpu.async_copy(zbuf.at[pl.ds(0, chunk)], acc.at[dst_v[1]],
                         ssems[1], add=True)

        plsc.subcore_barrier()

        prep(0, 0)

        def pair_body(i, _):
            cA = 2 * i
            wait_gather(0)
            wait_scatter(1)
            prep(cA + 1, 1)
            compute(0)
            scatter(0)
            wait_gather(1)
            wait_scatter(0)
            prep(cA + 2, 0)
            compute(1)
            scatter(1)
            return 0

        lax.fori_loop(0, (nchunks - 1) // 2, pair_body, 0)

        wait_gather(0)
        wait_scatter(1)
        compute(0)
        scatter(0)
        wait_scatter(0)

        plsc.subcore_barrier()

        for s in range(NSUB):
            @pl.when(sid == s)
            def _drain(s=s):
                for t in range(ndrain):
                    start = s * rows_per_tile + t * zrows
                    pltpu.sync_copy(acc.at[pl.ds(start, zrows)], zbuf)
                    pltpu.sync_copy(zbuf, out_hbm.at[cid, pl.ds(start, zrows)])

    return sc_edge


def kernel(x, edge_attr, W_in, b_in, W_node, b_node, W_edge, b_edge,
           W_msg, b_msg, ln_g, ln_b, edge_index):
    n, _ = x.shape
    e = edge_index.shape[1]
    ed = edge_attr.shape[1]
    hd = W_in.shape[1]
    nlayers = W_node.shape[0]

    wtop = W_msg[:, :hd, :]
    wbot = W_msg[:, hd:, :]
    wc = jnp.einsum("lij,ljk->lik", W_node, wtop)
    beta = (jnp.einsum("lj,ljk->lk", b_node, wtop)
            + jnp.einsum("lj,ljk->lk", b_edge, wbot) + b_msg)
    w2 = jnp.einsum("lij,ljk->lik", W_edge, wbot)

    src = edge_index[0]
    dst = edge_index[1]

    sc_edge = _make_sc_edge(n, e, hd, ed, 80)

    h, a = _in_mm(x, W_in, b_in, wc[0], beta[0])
    for l in range(nlayers):
        parts = sc_edge(a, edge_attr.reshape(-1), src, dst, w2[l])
        if l + 1 < nlayers:
            h, a = _ln_mm(h, parts[0, :n], parts[1, :n], ln_g[l], ln_b[l],
                          wc[l + 1], beta[l + 1])
        else:
            h = _ln_agg(h, parts[0, :n], parts[1, :n], ln_g[l], ln_b[l])
    return h
